# Initial kernel scaffold; baseline (speedup 1.0000x reference)
import jax
import jax.numpy as jnp
from jax.experimental import pallas as pl


def kernel(all_embeddings, edge_index, q, k, v):
    raise NotImplementedError("write your pallas kernel here")


# speedup vs baseline: 1.4448x; 1.4448x over previous
import jax
import jax.numpy as jnp
from jax import lax
from jax.experimental import pallas as pl
from jax.experimental.pallas import tpu as pltpu
from jax.experimental.pallas import tpu_sc as plsc

N = 10000
E = 320000
D = 128
H = 8
HD = D // H

NC = 2
NS = 16
L = 16

H2 = H // NC
DH = D // NC

C = 128
CHUNKS = E // C
BASE_CHUNKS = CHUNKS // NS
EXTRA = CHUNKS - BASE_CHUNKS * NS
NP = 10240
RPT = NP // NS
AW = DH + 16


def _proj_body(x_ref, wq_ref, wkv_ref, q_out, kv_out):
    x = x_ref[...]
    q_out[0] = jnp.dot(x, wq_ref[0], preferred_element_type=jnp.float32)
    kv_out[0] = jnp.dot(x, wkv_ref[0], preferred_element_type=jnp.float32)


def _project(emb, wq3, wkv3):
    blk = 400
    grid = (NC, N // blk)
    return pl.pallas_call(
        _proj_body,
        grid=grid,
        in_specs=[
            pl.BlockSpec((blk, D), lambda h, i: (i, 0)),
            pl.BlockSpec((1, D, DH), lambda h, i: (h, 0, 0)),
            pl.BlockSpec((1, D, 2 * DH), lambda h, i: (h, 0, 0)),
        ],
        out_specs=[
            pl.BlockSpec((1, blk, DH), lambda h, i: (h, i, 0)),
            pl.BlockSpec((1, blk, 2 * DH), lambda h, i: (h, i, 0)),
        ],
        out_shape=[
            jax.ShapeDtypeStruct((NC, N, DH), jnp.float32),
            jax.ShapeDtypeStruct((NC, N, 2 * DH), jnp.float32),
        ],
    )(emb, wq3, wkv3)


def _edge_body(qt, kvt, rows_h, cols_h, zacc,
               acc_out,
               rows_v, cols_v, rp_v, qbuf, kvbuf, obuf,
               acc_sh, sem_q, sem_kv):
    cid = lax.axis_index("c")
    sid = lax.axis_index("s")

    pltpu.sync_copy(zacc.at[pl.ds(sid * RPT, RPT)],
                    acc_sh.at[pl.ds(sid * RPT, RPT)])
    plsc.subcore_barrier()

    lane = lax.iota(jnp.int32, L)
    for pc in range(DH + H2, AW):
        for g0 in range(C // L):
            plsc.store_scatter(obuf,
                               [lane + g0 * L, jnp.full((L,), pc, jnp.int32)],
                               jnp.zeros((L,), jnp.float32))

    start = sid * BASE_CHUNKS + jnp.minimum(sid, EXTRA)
    cnt = BASE_CHUNKS + jnp.where(sid < EXTRA, 1, 0)

    cid_off = jnp.full((L,), 0, jnp.int32) + cid * N

    def chunk_body(j, carry):
        base = (start + j) * C
        pltpu.sync_copy(rows_h.at[pl.ds(base, C)], rows_v)
        pltpu.sync_copy(cols_h.at[pl.ds(base, C)], cols_v)
        for i in range(C // L):
            sl = pl.ds(i * L, L)
            rp_v[sl] = rows_v[sl] + cid_off
            cols_v[sl] = cols_v[sl] + cid_off
        cp_q = pltpu.async_copy(qt.at[rp_v], qbuf, sem_q)
        cp_kv = pltpu.async_copy(kvt.at[cols_v], kvbuf, sem_kv)
        cp_q.wait()
        cp_kv.wait()

        def group_body(g, carry2):
            erange = lane + g * L
            for h in range(H2):
                acc = jnp.zeros((L,), jnp.float32)
                for d in range(HD):
                    col = jnp.full((L,), h * HD + d, jnp.int32)
                    qv = plsc.load_gather(qbuf, [erange, col])
                    kv = plsc.load_gather(kvbuf, [erange, col])
                    acc = acc + qv * kv
                att = jnp.exp(jnp.clip(acc, -10.0, 10.0))
                plsc.store_scatter(obuf, [erange, jnp.full((L,), DH + h, jnp.int32)], att)
                for d in range(HD):
                    vcol = jnp.full((L,), DH + h * HD + d, jnp.int32)
                    vt = plsc.load_gather(kvbuf, [erange, vcol])
                    plsc.store_scatter(
                        obuf, [erange, jnp.full((L,), h * HD + d, jnp.int32)],
                        att * vt)
            return carry2

        lax.fori_loop(0, C // L, group_body, 0)

        pltpu.sync_copy(obuf, acc_sh.at[rows_v], add=True)
        return carry

    lax.fori_loop(0, cnt, chunk_body, 0)

    plsc.subcore_barrier()
    pltpu.sync_copy(acc_sh.at[pl.ds(sid * RPT, RPT)],
                    acc_out.at[cid, pl.ds(sid * RPT, RPT)])


def _edge_stage(qt, kvt, rows, cols, zacc):
    mesh = plsc.VectorSubcoreMesh(core_axis_name="c", subcore_axis_name="s",
                                  num_cores=NC, num_subcores=NS)
    f = pl.kernel(
        _edge_body,
        out_type=jax.ShapeDtypeStruct((NC, NP, AW), jnp.float32),
        mesh=mesh,
        scratch_types=(
            pltpu.VMEM((C,), jnp.int32),
            pltpu.VMEM((C,), jnp.int32),
            pltpu.VMEM((C,), jnp.int32),
            pltpu.VMEM((C, DH), jnp.float32),
            pltpu.VMEM((C, 2 * DH), jnp.float32),
            pltpu.VMEM((C, AW), jnp.float32),
            pltpu.VMEM_SHARED((NP, AW), jnp.float32),
            pltpu.SemaphoreType.DMA,
            pltpu.SemaphoreType.DMA,
        ),
        compiler_params=pltpu.CompilerParams(needs_layout_passes=False,
                                             use_tc_tiling_on_sc=False),
    )
    return f(qt, kvt, rows, cols, zacc)


def _final_body(acc_ref, s_ref, out_ref):
    na = jnp.dot(acc_ref[0, :, DH:DH + H2], s_ref[...],
                 preferred_element_type=jnp.float32)
    nb = jnp.dot(acc_ref[1, :, DH:DH + H2], s_ref[...],
                 preferred_element_type=jnp.float32)
    outa = acc_ref[0, :, :DH] / (na + 1e-8)
    outb = acc_ref[1, :, :DH] / (nb + 1e-8)
    out_ref[...] = jnp.concatenate([outa, outb], axis=1)


def _finalize(acc_p, s):
    blk = 400
    grid = (N // blk,)
    return pl.pallas_call(
        _final_body,
        grid=grid,
        in_specs=[
            pl.BlockSpec((NC, blk, AW), lambda i: (0, i, 0)),
            pl.BlockSpec((H2, DH), lambda i: (0, 0)),
        ],
        out_specs=pl.BlockSpec((blk, D), lambda i: (i, 0)),
        out_shape=jax.ShapeDtypeStruct((N, D), jnp.float32),
    )(acc_p, s)


@jax.jit
def kernel(all_embeddings, edge_index, q, k, v):
    wq3 = jnp.stack([q[:, :DH], q[:, DH:]])
    wkv3 = jnp.stack([
        jnp.concatenate([k[:, :DH], v[:, :DH]], axis=1),
        jnp.concatenate([k[:, DH:], v[:, DH:]], axis=1),
    ])
    qt, kvt = _project(all_embeddings, wq3, wkv3)
    qt2 = qt.reshape(NC * N, DH)
    kvt2 = kvt.reshape(NC * N, 2 * DH)
    rows = edge_index[0]
    cols = edge_index[1]
    zacc = jnp.zeros((NP, AW), jnp.float32)
    acc_p = _edge_stage(qt2, kvt2, rows, cols, zacc)
    s = jnp.repeat(jnp.eye(H2, dtype=jnp.float32), HD, axis=1)
    return _finalize(acc_p, s)


# speedup vs baseline: 1.4762x; 1.0217x over previous
import jax
import jax.numpy as jnp
from jax import lax
from jax.experimental import pallas as pl
from jax.experimental.pallas import tpu as pltpu
from jax.experimental.pallas import tpu_sc as plsc

N = 10000
E = 320000
D = 128
H = 8
HD = D // H

NC = 2
NS = 16
L = 16

H2 = H // NC
DH = D // NC

C = 128
CHUNKS = E // C
BASE_CHUNKS = CHUNKS // NS
EXTRA = CHUNKS - BASE_CHUNKS * NS
NP = 10240
RPT = NP // NS
AW = DH + 16


def _proj_body(x_ref, wq_ref, wkv_ref, q_out, kv_out):
    x = x_ref[...]
    q_out[0] = jnp.dot(x, wq_ref[0], preferred_element_type=jnp.float32)
    kv_out[0] = jnp.dot(x, wkv_ref[0], preferred_element_type=jnp.float32)


def _project(emb, wq3, wkv3):
    blk = 400
    grid = (NC, N // blk)
    return pl.pallas_call(
        _proj_body,
        grid=grid,
        in_specs=[
            pl.BlockSpec((blk, D), lambda h, i: (i, 0)),
            pl.BlockSpec((1, D, DH), lambda h, i: (h, 0, 0)),
            pl.BlockSpec((1, D, 2 * DH), lambda h, i: (h, 0, 0)),
        ],
        out_specs=[
            pl.BlockSpec((1, blk, DH), lambda h, i: (h, i, 0)),
            pl.BlockSpec((1, blk, 2 * DH), lambda h, i: (h, i, 0)),
        ],
        out_shape=[
            jax.ShapeDtypeStruct((NC, N, DH), jnp.float32),
            jax.ShapeDtypeStruct((NC, N, 2 * DH), jnp.float32),
        ],
    )(emb, wq3, wkv3)


def _edge_body(qt, kvt, rows_h, cols_h, zacc,
               acc_out,
               rows_v, cols_v, rp_v, qbuf, kvbuf, obuf,
               acc_sh, sem_q, sem_kv):
    cid = lax.axis_index("c")
    sid = lax.axis_index("s")

    pltpu.sync_copy(zacc.at[pl.ds(sid * RPT, RPT)],
                    acc_sh.at[pl.ds(sid * RPT, RPT)])
    plsc.subcore_barrier()

    lane = lax.iota(jnp.int32, L)
    for pc in range(DH + H2, AW):
        for g0 in range(C // L):
            plsc.store_scatter(obuf,
                               [lane + g0 * L, jnp.full((L,), pc, jnp.int32)],
                               jnp.zeros((L,), jnp.float32))

    start = sid * BASE_CHUNKS + jnp.minimum(sid, EXTRA)
    cnt = BASE_CHUNKS + jnp.where(sid < EXTRA, 1, 0)

    cid_off = jnp.full((L,), 0, jnp.int32) + cid * N

    def chunk_body(j, carry):
        base = (start + j) * C
        pltpu.sync_copy(rows_h.at[pl.ds(base, C)], rows_v)
        pltpu.sync_copy(cols_h.at[pl.ds(base, C)], cols_v)
        for i in range(C // L):
            sl = pl.ds(i * L, L)
            rp_v[sl] = rows_v[sl] + cid_off
            cols_v[sl] = cols_v[sl] + cid_off
        cp_q = pltpu.async_copy(qt.at[rp_v], qbuf, sem_q)
        cp_kv = pltpu.async_copy(kvt.at[cols_v], kvbuf, sem_kv)
        cp_q.wait()
        cp_kv.wait()

        @plsc.parallel_loop(0, C // L, unroll=2)
        def group_body(g):
            erange = lane + g * L
            for h in range(H2):
                parts = [jnp.zeros((L,), jnp.float32) for _ in range(4)]
                for d in range(HD):
                    col = jnp.full((L,), h * HD + d, jnp.int32)
                    qv = plsc.load_gather(qbuf, [erange, col])
                    kv = plsc.load_gather(kvbuf, [erange, col])
                    parts[d % 4] = parts[d % 4] + qv * kv
                acc = (parts[0] + parts[1]) + (parts[2] + parts[3])
                att = jnp.exp(jnp.clip(acc, -10.0, 10.0))
                plsc.store_scatter(obuf, [erange, jnp.full((L,), DH + h, jnp.int32)], att)
                for d in range(HD):
                    vcol = jnp.full((L,), DH + h * HD + d, jnp.int32)
                    vt = plsc.load_gather(kvbuf, [erange, vcol])
                    plsc.store_scatter(
                        obuf, [erange, jnp.full((L,), h * HD + d, jnp.int32)],
                        att * vt)

        pltpu.sync_copy(obuf, acc_sh.at[rows_v], add=True)
        return carry

    lax.fori_loop(0, cnt, chunk_body, 0)

    plsc.subcore_barrier()
    pltpu.sync_copy(acc_sh.at[pl.ds(sid * RPT, RPT)],
                    acc_out.at[cid, pl.ds(sid * RPT, RPT)])


def _edge_stage(qt, kvt, rows, cols, zacc):
    mesh = plsc.VectorSubcoreMesh(core_axis_name="c", subcore_axis_name="s",
                                  num_cores=NC, num_subcores=NS)
    f = pl.kernel(
        _edge_body,
        out_type=jax.ShapeDtypeStruct((NC, NP, AW), jnp.float32),
        mesh=mesh,
        scratch_types=(
            pltpu.VMEM((C,), jnp.int32),
            pltpu.VMEM((C,), jnp.int32),
            pltpu.VMEM((C,), jnp.int32),
            pltpu.VMEM((C, DH), jnp.float32),
            pltpu.VMEM((C, 2 * DH), jnp.float32),
            pltpu.VMEM((C, AW), jnp.float32),
            pltpu.VMEM_SHARED((NP, AW), jnp.float32),
            pltpu.SemaphoreType.DMA,
            pltpu.SemaphoreType.DMA,
        ),
        compiler_params=pltpu.CompilerParams(needs_layout_passes=False,
                                             use_tc_tiling_on_sc=False),
    )
    return f(qt, kvt, rows, cols, zacc)


def _final_body(acc_ref, s_ref, out_ref):
    na = jnp.dot(acc_ref[0, :, DH:DH + H2], s_ref[...],
                 preferred_element_type=jnp.float32)
    nb = jnp.dot(acc_ref[1, :, DH:DH + H2], s_ref[...],
                 preferred_element_type=jnp.float32)
    outa = acc_ref[0, :, :DH] / (na + 1e-8)
    outb = acc_ref[1, :, :DH] / (nb + 1e-8)
    out_ref[...] = jnp.concatenate([outa, outb], axis=1)


def _finalize(acc_p, s):
    blk = 400
    grid = (N // blk,)
    return pl.pallas_call(
        _final_body,
        grid=grid,
        in_specs=[
            pl.BlockSpec((NC, blk, AW), lambda i: (0, i, 0)),
            pl.BlockSpec((H2, DH), lambda i: (0, 0)),
        ],
        out_specs=pl.BlockSpec((blk, D), lambda i: (i, 0)),
        out_shape=jax.ShapeDtypeStruct((N, D), jnp.float32),
    )(acc_p, s)


@jax.jit
def kernel(all_embeddings, edge_index, q, k, v):
    wq3 = jnp.stack([q[:, :DH], q[:, DH:]])
    wkv3 = jnp.stack([
        jnp.concatenate([k[:, :DH], v[:, :DH]], axis=1),
        jnp.concatenate([k[:, DH:], v[:, DH:]], axis=1),
    ])
    qt, kvt = _project(all_embeddings, wq3, wkv3)
    qt2 = qt.reshape(NC * N, DH)
    kvt2 = kvt.reshape(NC * N, 2 * DH)
    rows = edge_index[0]
    cols = edge_index[1]
    zacc = jnp.zeros((NP, AW), jnp.float32)
    acc_p = _edge_stage(qt2, kvt2, rows, cols, zacc)
    s = jnp.repeat(jnp.eye(H2, dtype=jnp.float32), HD, axis=1)
    return _finalize(acc_p, s)


# speedup vs baseline: 2.1100x; 1.4294x over previous
import jax
import jax.numpy as jnp
from jax import lax
from jax.experimental import pallas as pl
from jax.experimental.pallas import tpu as pltpu
from jax.experimental.pallas import tpu_sc as plsc

N = 10000
E = 320000
D = 128
H = 8
HD = D // H

NC = 2
NS = 16

C = 128
CHUNKS = E // C
W = NC * NS
GBASE = CHUNKS // W
GEXTRA = CHUNKS - GBASE * W

CPC = CHUNKS // NC
SBASE = CPC // NS
SEXTRA = CPC - SBASE * NS

NP = 10240
RPT = NP // NS
PW = D + 16

EB = 512
NB = 400


def _proj_body(x_ref, wq_ref, wkv_ref, q_out, kv_out):
    x = x_ref[...]
    q_out[...] = jnp.dot(x, wq_ref[...], preferred_element_type=jnp.float32)
    kv_out[...] = jnp.dot(x, wkv_ref[...], preferred_element_type=jnp.float32)


def _project(emb, wq, wkv):
    grid = (N // NB,)
    return pl.pallas_call(
        _proj_body,
        grid=grid,
        in_specs=[
            pl.BlockSpec((NB, D), lambda i: (i, 0)),
            pl.BlockSpec((D, D), lambda i: (0, 0)),
            pl.BlockSpec((D, 2 * D), lambda i: (0, 0)),
        ],
        out_specs=[
            pl.BlockSpec((NB, D), lambda i: (i, 0)),
            pl.BlockSpec((NB, 2 * D), lambda i: (i, 0)),
        ],
        out_shape=[
            jax.ShapeDtypeStruct((N, D), jnp.float32),
            jax.ShapeDtypeStruct((N, 2 * D), jnp.float32),
        ],
    )(emb, wq, wkv)


def _gather_body(qt, kvt, rows_h, cols_h,
                 qe_out, kve_out,
                 rows_v, cols_v, qbuf, kvbuf, sem_q, sem_kv):
    cid = lax.axis_index("c")
    sid = lax.axis_index("s")
    w = cid * NS + sid
    start = w * GBASE + jnp.minimum(w, GEXTRA)
    cnt = GBASE + jnp.where(w < GEXTRA, 1, 0)

    def chunk_body(j, carry):
        base = (start + j) * C
        pltpu.sync_copy(rows_h.at[pl.ds(base, C)], rows_v)
        pltpu.sync_copy(cols_h.at[pl.ds(base, C)], cols_v)
        cp_q = pltpu.async_copy(qt.at[rows_v], qbuf, sem_q)
        cp_kv = pltpu.async_copy(kvt.at[cols_v], kvbuf, sem_kv)
        cp_q.wait()
        cp_kv.wait()
        pltpu.sync_copy(qbuf, qe_out.at[pl.ds(base, C)])
        pltpu.sync_copy(kvbuf, kve_out.at[pl.ds(base, C)])
        return carry

    lax.fori_loop(0, cnt, chunk_body, 0)


def _gather_stage(qt, kvt, rows, cols):
    mesh = plsc.VectorSubcoreMesh(core_axis_name="c", subcore_axis_name="s",
                                  num_cores=NC, num_subcores=NS)
    f = pl.kernel(
        _gather_body,
        out_type=[
            jax.ShapeDtypeStruct((E, D), jnp.float32),
            jax.ShapeDtypeStruct((E, 2 * D), jnp.float32),
        ],
        mesh=mesh,
        scratch_types=(
            pltpu.VMEM((C,), jnp.int32),
            pltpu.VMEM((C,), jnp.int32),
            pltpu.VMEM((C, D), jnp.float32),
            pltpu.VMEM((C, 2 * D), jnp.float32),
            pltpu.SemaphoreType.DMA,
            pltpu.SemaphoreType.DMA,
        ),
        compiler_params=pltpu.CompilerParams(needs_layout_passes=False,
                                             use_tc_tiling_on_sc=False),
    )
    return f(qt, kvt, rows, cols)


def _edge_math_body(qe_ref, kve_ref, m1_ref, pay_ref):
    qe = qe_ref[...]
    ke = kve_ref[:, :D]
    ve = kve_ref[:, D:]
    m1 = m1_ref[...]
    s = jnp.dot(qe * ke, m1, preferred_element_type=jnp.float32)
    att = jnp.exp(jnp.clip(s, -10.0, 10.0))
    attb = jnp.dot(att, m1.T, preferred_element_type=jnp.float32)
    pay_ref[:, :D] = attb * ve
    pay_ref[:, D:D + H] = att
    pay_ref[:, D + H:] = jnp.zeros((EB, PW - D - H), jnp.float32)


def _edge_math(qe, kve, m1):
    grid = (E // EB,)
    return pl.pallas_call(
        _edge_math_body,
        grid=grid,
        in_specs=[
            pl.BlockSpec((EB, D), lambda i: (i, 0)),
            pl.BlockSpec((EB, 2 * D), lambda i: (i, 0)),
            pl.BlockSpec((D, H), lambda i: (0, 0)),
        ],
        out_specs=pl.BlockSpec((EB, PW), lambda i: (i, 0)),
        out_shape=jax.ShapeDtypeStruct((E, PW), jnp.float32),
    )(qe, kve, m1)


def _scatter_body(pay, rows_h, zacc,
                  acc_out,
                  rows_v, pbuf, acc_sh, sem_p):
    cid = lax.axis_index("c")
    sid = lax.axis_index("s")

    pltpu.sync_copy(zacc.at[pl.ds(sid * RPT, RPT)],
                    acc_sh.at[pl.ds(sid * RPT, RPT)])
    plsc.subcore_barrier()

    start = cid * CPC + sid * SBASE + jnp.minimum(sid, SEXTRA)
    cnt = SBASE + jnp.where(sid < SEXTRA, 1, 0)

    def chunk_body(j, carry):
        base = (start + j) * C
        pltpu.sync_copy(rows_h.at[pl.ds(base, C)], rows_v)
        cp = pltpu.async_copy(pay.at[pl.ds(base, C)], pbuf, sem_p)
        cp.wait()
        pltpu.sync_copy(pbuf, acc_sh.at[rows_v], add=True)
        return carry

    lax.fori_loop(0, cnt, chunk_body, 0)

    plsc.subcore_barrier()
    pltpu.sync_copy(acc_sh.at[pl.ds(sid * RPT, RPT)],
                    acc_out.at[cid, pl.ds(sid * RPT, RPT)])


def _scatter_stage(pay, rows, zacc):
    mesh = plsc.VectorSubcoreMesh(core_axis_name="c", subcore_axis_name="s",
                                  num_cores=NC, num_subcores=NS)
    f = pl.kernel(
        _scatter_body,
        out_type=jax.ShapeDtypeStruct((NC, NP, PW), jnp.float32),
        mesh=mesh,
        scratch_types=(
            pltpu.VMEM((C,), jnp.int32),
            pltpu.VMEM((C, PW), jnp.float32),
            pltpu.VMEM_SHARED((NP, PW), jnp.float32),
            pltpu.SemaphoreType.DMA,
        ),
        compiler_params=pltpu.CompilerParams(needs_layout_passes=False,
                                             use_tc_tiling_on_sc=False),
    )
    return f(pay, rows, zacc)


def _final_body(acc_ref, m2_ref, out_ref):
    acc = acc_ref[0] + acc_ref[1]
    den = jnp.dot(acc[:, D:D + H], m2_ref[...],
                  preferred_element_type=jnp.float32)
    out_ref[...] = acc[:, :D] / (den + 1e-8)


def _finalize(acc_p, m2):
    grid = (N // NB,)
    return pl.pallas_call(
        _final_body,
        grid=grid,
        in_specs=[
            pl.BlockSpec((NC, NB, PW), lambda i: (0, i, 0)),
            pl.BlockSpec((H, D), lambda i: (0, 0)),
        ],
        out_specs=pl.BlockSpec((NB, D), lambda i: (i, 0)),
        out_shape=jax.ShapeDtypeStruct((N, D), jnp.float32),
    )(acc_p, m2)


@jax.jit
def kernel(all_embeddings, edge_index, q, k, v):
    wkv = jnp.concatenate([k, v], axis=1)
    qt, kvt = _project(all_embeddings, q, wkv)
    rows = edge_index[0]
    cols = edge_index[1]
    qe, kve = _gather_stage(qt, kvt, rows, cols)
    m1 = jnp.repeat(jnp.eye(H, dtype=jnp.float32), HD, axis=0)
    pay = _edge_math(qe, kve, m1)
    zacc = jnp.zeros((NP, PW), jnp.float32)
    acc_p = _scatter_stage(pay, rows, zacc)
    m2 = jnp.repeat(jnp.eye(H, dtype=jnp.float32), HD, axis=1)
    return _finalize(acc_p, m2)


# speedup vs baseline: 2.3143x; 1.0968x over previous
import jax
import jax.numpy as jnp
from jax import lax
from jax.experimental import pallas as pl
from jax.experimental.pallas import tpu as pltpu
from jax.experimental.pallas import tpu_sc as plsc

N = 10000
E = 320000
D = 128
H = 8
HD = D // H

NC = 2
NS = 16

C = 128
CHUNKS = E // C
W = NC * NS
GBASE = CHUNKS // W
GEXTRA = CHUNKS - GBASE * W

CPC = CHUNKS // NC
SBASE = CPC // NS
SEXTRA = CPC - SBASE * NS

NP = 10240
RPT = NP // NS
PW = D + 16

EB = 1024
NB = 400


def _proj_body(x_ref, wq_ref, wkv_ref, q_out, kv_out):
    x = x_ref[...]
    q_out[...] = jnp.dot(x, wq_ref[...], preferred_element_type=jnp.float32)
    kv_out[...] = jnp.dot(x, wkv_ref[...], preferred_element_type=jnp.float32)


def _project(emb, wq, wkv):
    grid = (N // NB,)
    return pl.pallas_call(
        _proj_body,
        grid=grid,
        in_specs=[
            pl.BlockSpec((NB, D), lambda i: (i, 0)),
            pl.BlockSpec((D, D), lambda i: (0, 0)),
            pl.BlockSpec((D, 2 * D), lambda i: (0, 0)),
        ],
        out_specs=[
            pl.BlockSpec((NB, D), lambda i: (i, 0)),
            pl.BlockSpec((NB, 2 * D), lambda i: (i, 0)),
        ],
        out_shape=[
            jax.ShapeDtypeStruct((N, D), jnp.float32),
            jax.ShapeDtypeStruct((N, 2 * D), jnp.float32),
        ],
    )(emb, wq, wkv)


def _gather_body(qt, kvt, rows_h, cols_h,
                 qe_out, kve_out,
                 rows_v, cols_v, qbuf, kvbuf, sem_q, sem_kv):
    cid = lax.axis_index("c")
    sid = lax.axis_index("s")
    w = cid * NS + sid
    start = w * GBASE + jnp.minimum(w, GEXTRA)
    cnt = GBASE + jnp.where(w < GEXTRA, 1, 0)

    def chunk_body(j, carry):
        base = (start + j) * C
        pltpu.sync_copy(rows_h.at[pl.ds(base, C)], rows_v)
        pltpu.sync_copy(cols_h.at[pl.ds(base, C)], cols_v)
        cp_q = pltpu.async_copy(qt.at[rows_v], qbuf, sem_q)
        cp_kv = pltpu.async_copy(kvt.at[cols_v], kvbuf, sem_kv)
        cp_q.wait()
        cp_kv.wait()
        pltpu.sync_copy(qbuf, qe_out.at[pl.ds(base, C)])
        pltpu.sync_copy(kvbuf, kve_out.at[pl.ds(base, C)])
        return carry

    lax.fori_loop(0, cnt, chunk_body, 0)


def _gather_stage(qt, kvt, rows, cols):
    mesh = plsc.VectorSubcoreMesh(core_axis_name="c", subcore_axis_name="s",
                                  num_cores=NC, num_subcores=NS)
    f = pl.kernel(
        _gather_body,
        out_type=[
            jax.ShapeDtypeStruct((E, D), jnp.float32),
            jax.ShapeDtypeStruct((E, 2 * D), jnp.float32),
        ],
        mesh=mesh,
        scratch_types=(
            pltpu.VMEM((C,), jnp.int32),
            pltpu.VMEM((C,), jnp.int32),
            pltpu.VMEM((C, D), jnp.float32),
            pltpu.VMEM((C, 2 * D), jnp.float32),
            pltpu.SemaphoreType.DMA,
            pltpu.SemaphoreType.DMA,
        ),
        compiler_params=pltpu.CompilerParams(needs_layout_passes=False,
                                             use_tc_tiling_on_sc=False),
    )
    return f(qt, kvt, rows, cols)


def _edge_math_body(qe_ref, kve_ref, m1_ref, pay_ref):
    qe = qe_ref[...]
    ke = kve_ref[:, :D]
    ve = kve_ref[:, D:]
    m1 = m1_ref[...]
    s = jnp.dot(qe * ke, m1, preferred_element_type=jnp.float32)
    att = jnp.exp(jnp.clip(s, -10.0, 10.0))
    attb = jnp.dot(att, m1.T, preferred_element_type=jnp.float32)
    pay_ref[:, :D] = attb * ve
    pay_ref[:, D:D + H] = att
    pay_ref[:, D + H:] = jnp.zeros((EB, PW - D - H), jnp.float32)


def _edge_math(qe, kve, m1):
    grid = (E // EB,)
    return pl.pallas_call(
        _edge_math_body,
        grid=grid,
        in_specs=[
            pl.BlockSpec((EB, D), lambda i: (i, 0)),
            pl.BlockSpec((EB, 2 * D), lambda i: (i, 0)),
            pl.BlockSpec((D, H), lambda i: (0, 0)),
        ],
        out_specs=pl.BlockSpec((EB, PW), lambda i: (i, 0)),
        out_shape=jax.ShapeDtypeStruct((E, PW), jnp.float32),
    )(qe, kve, m1)


def _scatter_body(pay, rows_h, zacc,
                  acc_out,
                  rows_v, pbuf, acc_sh, sem_p):
    cid = lax.axis_index("c")
    sid = lax.axis_index("s")

    pltpu.sync_copy(zacc.at[pl.ds(sid * RPT, RPT)],
                    acc_sh.at[pl.ds(sid * RPT, RPT)])
    plsc.subcore_barrier()

    start = cid * CPC + sid * SBASE + jnp.minimum(sid, SEXTRA)
    cnt = SBASE + jnp.where(sid < SEXTRA, 1, 0)

    def chunk_body(j, carry):
        base = (start + j) * C
        pltpu.sync_copy(rows_h.at[pl.ds(base, C)], rows_v)
        cp = pltpu.async_copy(pay.at[pl.ds(base, C)], pbuf, sem_p)
        cp.wait()
        pltpu.sync_copy(pbuf, acc_sh.at[rows_v], add=True)
        return carry

    lax.fori_loop(0, cnt, chunk_body, 0)

    plsc.subcore_barrier()
    pltpu.sync_copy(acc_sh.at[pl.ds(sid * RPT, RPT)],
                    acc_out.at[cid, pl.ds(sid * RPT, RPT)])


def _scatter_stage(pay, rows, zacc):
    mesh = plsc.VectorSubcoreMesh(core_axis_name="c", subcore_axis_name="s",
                                  num_cores=NC, num_subcores=NS)
    f = pl.kernel(
        _scatter_body,
        out_type=jax.ShapeDtypeStruct((NC, NP, PW), jnp.float32),
        mesh=mesh,
        scratch_types=(
            pltpu.VMEM((C,), jnp.int32),
            pltpu.VMEM((C, PW), jnp.float32),
            pltpu.VMEM_SHARED((NP, PW), jnp.float32),
            pltpu.SemaphoreType.DMA,
        ),
        compiler_params=pltpu.CompilerParams(needs_layout_passes=False,
                                             use_tc_tiling_on_sc=False),
    )
    return f(pay, rows, zacc)


def _final_body(acc_ref, m2_ref, out_ref):
    acc = acc_ref[0] + acc_ref[1]
    den = jnp.dot(acc[:, D:D + H], m2_ref[...],
                  preferred_element_type=jnp.float32)
    out_ref[...] = acc[:, :D] / (den + 1e-8)


def _finalize(acc_p, m2):
    grid = (N // NB,)
    return pl.pallas_call(
        _final_body,
        grid=grid,
        in_specs=[
            pl.BlockSpec((NC, NB, PW), lambda i: (0, i, 0)),
            pl.BlockSpec((H, D), lambda i: (0, 0)),
        ],
        out_specs=pl.BlockSpec((NB, D), lambda i: (i, 0)),
        out_shape=jax.ShapeDtypeStruct((N, D), jnp.float32),
    )(acc_p, m2)


@jax.jit
def kernel(all_embeddings, edge_index, q, k, v):
    wkv = jnp.concatenate([k, v], axis=1)
    qt, kvt = _project(all_embeddings, q, wkv)
    rows = edge_index[0]
    cols = edge_index[1]
    qe, kve = _gather_stage(qt, kvt, rows, cols)
    m1 = jnp.repeat(jnp.eye(H, dtype=jnp.float32), HD, axis=0)
    pay = _edge_math(qe, kve, m1)
    zacc = jnp.zeros((NP, PW), jnp.float32)
    acc_p = _scatter_stage(pay, rows, zacc)
    m2 = jnp.repeat(jnp.eye(H, dtype=jnp.float32), HD, axis=1)
    return _finalize(acc_p, m2)


# speedup vs baseline: 2.3717x; 1.0248x over previous
import jax
import jax.numpy as jnp
from jax import lax
from jax.experimental import pallas as pl
from jax.experimental.pallas import tpu as pltpu
from jax.experimental.pallas import tpu_sc as plsc

N = 10000
E = 320000
D = 128
H = 8
HD = D // H

NC = 2
NS = 16

C = 128
CHUNKS = E // C
W = NC * NS
GBASE = CHUNKS // W
GEXTRA = CHUNKS - GBASE * W

CPC = CHUNKS // NC
SBASE = CPC // NS
SEXTRA = CPC - SBASE * NS

NP = 10240
RPT = NP // NS
PW = D + 16

EB = 1280
NB = 400


def _proj_body(x_ref, wq_ref, wkv_ref, q_out, kv_out):
    x = x_ref[...]
    q_out[...] = jnp.dot(x, wq_ref[...], preferred_element_type=jnp.float32)
    kv_out[...] = jnp.dot(x, wkv_ref[...], preferred_element_type=jnp.float32)


def _project(emb, wq, wkv):
    grid = (N // NB,)
    return pl.pallas_call(
        _proj_body,
        grid=grid,
        in_specs=[
            pl.BlockSpec((NB, D), lambda i: (i, 0)),
            pl.BlockSpec((D, D), lambda i: (0, 0)),
            pl.BlockSpec((D, 2 * D), lambda i: (0, 0)),
        ],
        out_specs=[
            pl.BlockSpec((NB, D), lambda i: (i, 0)),
            pl.BlockSpec((NB, 2 * D), lambda i: (i, 0)),
        ],
        out_shape=[
            jax.ShapeDtypeStruct((N, D), jnp.float32),
            jax.ShapeDtypeStruct((N, 2 * D), jnp.float32),
        ],
    )(emb, wq, wkv)


def _gather_body(qt, kvt, rows_h, cols_h,
                 qe_out, kve_out,
                 rows_v, cols_v, qbuf, kvbuf, sem_q, sem_kv):
    cid = lax.axis_index("c")
    sid = lax.axis_index("s")
    w = cid * NS + sid
    start = w * GBASE + jnp.minimum(w, GEXTRA)
    cnt = GBASE + jnp.where(w < GEXTRA, 1, 0)

    def chunk_body(j, carry):
        base = (start + j) * C
        pltpu.sync_copy(rows_h.at[pl.ds(base, C)], rows_v)
        pltpu.sync_copy(cols_h.at[pl.ds(base, C)], cols_v)
        cp_q = pltpu.async_copy(qt.at[rows_v], qbuf, sem_q)
        cp_kv = pltpu.async_copy(kvt.at[cols_v], kvbuf, sem_kv)
        cp_q.wait()
        cp_kv.wait()
        pltpu.sync_copy(qbuf, qe_out.at[pl.ds(base, C)])
        pltpu.sync_copy(kvbuf, kve_out.at[pl.ds(base, C)])
        return carry

    lax.fori_loop(0, cnt, chunk_body, 0)


def _gather_stage(qt, kvt, rows, cols):
    mesh = plsc.VectorSubcoreMesh(core_axis_name="c", subcore_axis_name="s",
                                  num_cores=NC, num_subcores=NS)
    f = pl.kernel(
        _gather_body,
        out_type=[
            jax.ShapeDtypeStruct((E, D), jnp.float32),
            jax.ShapeDtypeStruct((E, 2 * D), jnp.float32),
        ],
        mesh=mesh,
        scratch_types=(
            pltpu.VMEM((C,), jnp.int32),
            pltpu.VMEM((C,), jnp.int32),
            pltpu.VMEM((C, D), jnp.float32),
            pltpu.VMEM((C, 2 * D), jnp.float32),
            pltpu.SemaphoreType.DMA,
            pltpu.SemaphoreType.DMA,
        ),
        compiler_params=pltpu.CompilerParams(needs_layout_passes=False,
                                             use_tc_tiling_on_sc=False),
    )
    return f(qt, kvt, rows, cols)


def _edge_math_body(qe_ref, kve_ref, m1_ref, pay_ref):
    qe = qe_ref[...]
    ke = kve_ref[:, :D]
    ve = kve_ref[:, D:]
    m1 = m1_ref[...]
    s = jnp.dot(qe * ke, m1, preferred_element_type=jnp.float32)
    att = jnp.exp(jnp.clip(s, -10.0, 10.0))
    attb = jnp.dot(att, m1.T, preferred_element_type=jnp.float32)
    pay_ref[:, :D] = attb * ve
    pay_ref[:, D:D + H] = att
    pay_ref[:, D + H:] = jnp.zeros((EB, PW - D - H), jnp.float32)


def _edge_math(qe, kve, m1):
    grid = (E // EB,)
    return pl.pallas_call(
        _edge_math_body,
        grid=grid,
        in_specs=[
            pl.BlockSpec((EB, D), lambda i: (i, 0)),
            pl.BlockSpec((EB, 2 * D), lambda i: (i, 0)),
            pl.BlockSpec((D, H), lambda i: (0, 0)),
        ],
        out_specs=pl.BlockSpec((EB, PW), lambda i: (i, 0)),
        out_shape=jax.ShapeDtypeStruct((E, PW), jnp.float32),
    )(qe, kve, m1)


def _scatter_body(pay, rows_h, zacc,
                  acc_out,
                  rows_v, pbuf, acc_sh, sem_p):
    cid = lax.axis_index("c")
    sid = lax.axis_index("s")

    pltpu.sync_copy(zacc.at[pl.ds(sid * RPT, RPT)],
                    acc_sh.at[pl.ds(sid * RPT, RPT)])
    plsc.subcore_barrier()

    start = cid * CPC + sid * SBASE + jnp.minimum(sid, SEXTRA)
    cnt = SBASE + jnp.where(sid < SEXTRA, 1, 0)

    def chunk_body(j, carry):
        base = (start + j) * C
        pltpu.sync_copy(rows_h.at[pl.ds(base, C)], rows_v)
        cp = pltpu.async_copy(pay.at[pl.ds(base, C)], pbuf, sem_p)
        cp.wait()
        pltpu.sync_copy(pbuf, acc_sh.at[rows_v], add=True)
        return carry

    lax.fori_loop(0, cnt, chunk_body, 0)

    plsc.subcore_barrier()
    pltpu.sync_copy(acc_sh.at[pl.ds(sid * RPT, RPT)],
                    acc_out.at[cid, pl.ds(sid * RPT, RPT)])


def _scatter_stage(pay, rows, zacc):
    mesh = plsc.VectorSubcoreMesh(core_axis_name="c", subcore_axis_name="s",
                                  num_cores=NC, num_subcores=NS)
    f = pl.kernel(
        _scatter_body,
        out_type=jax.ShapeDtypeStruct((NC, NP, PW), jnp.float32),
        mesh=mesh,
        scratch_types=(
            pltpu.VMEM((C,), jnp.int32),
            pltpu.VMEM((C, PW), jnp.float32),
            pltpu.VMEM_SHARED((NP, PW), jnp.float32),
            pltpu.SemaphoreType.DMA,
        ),
        compiler_params=pltpu.CompilerParams(needs_layout_passes=False,
                                             use_tc_tiling_on_sc=False),
    )
    return f(pay, rows, zacc)


def _final_body(acc_ref, m2_ref, out_ref):
    acc = acc_ref[0] + acc_ref[1]
    den = jnp.dot(acc[:, D:D + H], m2_ref[...],
                  preferred_element_type=jnp.float32)
    out_ref[...] = acc[:, :D] / (den + 1e-8)


def _finalize(acc_p, m2):
    grid = (N // NB,)
    return pl.pallas_call(
        _final_body,
        grid=grid,
        in_specs=[
            pl.BlockSpec((NC, NB, PW), lambda i: (0, i, 0)),
            pl.BlockSpec((H, D), lambda i: (0, 0)),
        ],
        out_specs=pl.BlockSpec((NB, D), lambda i: (i, 0)),
        out_shape=jax.ShapeDtypeStruct((N, D), jnp.float32),
    )(acc_p, m2)


@jax.jit
def kernel(all_embeddings, edge_index, q, k, v):
    wkv = jnp.concatenate([k, v], axis=1)
    qt, kvt = _project(all_embeddings, q, wkv)
    rows = edge_index[0]
    cols = edge_index[1]
    qe, kve = _gather_stage(qt, kvt, rows, cols)
    m1 = jnp.repeat(jnp.eye(H, dtype=jnp.float32), HD, axis=0)
    pay = _edge_math(qe, kve, m1)
    zacc = jnp.zeros((NP, PW), jnp.float32)
    acc_p = _scatter_stage(pay, rows, zacc)
    m2 = jnp.repeat(jnp.eye(H, dtype=jnp.float32), HD, axis=1)
    return _finalize(acc_p, m2)


# speedup vs baseline: 2.4653x; 1.0395x over previous
import jax
import jax.numpy as jnp
from jax import lax
from jax.experimental import pallas as pl
from jax.experimental.pallas import tpu as pltpu
from jax.experimental.pallas import tpu_sc as plsc

N = 10000
E = 320000
D = 128
H = 8
HD = D // H

NC = 2
NS = 16

C = 128
CHUNKS = E // C
W = NC * NS
GBASE = CHUNKS // W
GEXTRA = CHUNKS - GBASE * W

CPC = CHUNKS // NC
SBASE = CPC // NS
SEXTRA = CPC - SBASE * NS

NP = 10240
RPT = NP // NS
PW = D + 16

EB = 2560
NB = 400


def _proj_body(x_ref, wq_ref, wkv_ref, q_out, kv_out):
    x = x_ref[...]
    q_out[...] = jnp.dot(x, wq_ref[...], preferred_element_type=jnp.float32)
    kv_out[...] = jnp.dot(x, wkv_ref[...], preferred_element_type=jnp.float32)


def _project(emb, wq, wkv):
    grid = (N // NB,)
    return pl.pallas_call(
        _proj_body,
        grid=grid,
        in_specs=[
            pl.BlockSpec((NB, D), lambda i: (i, 0)),
            pl.BlockSpec((D, D), lambda i: (0, 0)),
            pl.BlockSpec((D, 2 * D), lambda i: (0, 0)),
        ],
        out_specs=[
            pl.BlockSpec((NB, D), lambda i: (i, 0)),
            pl.BlockSpec((NB, 2 * D), lambda i: (i, 0)),
        ],
        out_shape=[
            jax.ShapeDtypeStruct((N, D), jnp.float32),
            jax.ShapeDtypeStruct((N, 2 * D), jnp.float32),
        ],
    )(emb, wq, wkv)


def _gather_body(qt, kvt, rows_h, cols_h,
                 qe_out, kve_out,
                 rows_v, cols_v, qbuf, kvbuf, sem_q, sem_kv):
    cid = lax.axis_index("c")
    sid = lax.axis_index("s")
    w = cid * NS + sid
    start = w * GBASE + jnp.minimum(w, GEXTRA)
    cnt = GBASE + jnp.where(w < GEXTRA, 1, 0)

    def chunk_body(j, carry):
        base = (start + j) * C
        pltpu.sync_copy(rows_h.at[pl.ds(base, C)], rows_v)
        pltpu.sync_copy(cols_h.at[pl.ds(base, C)], cols_v)
        cp_q = pltpu.async_copy(qt.at[rows_v], qbuf, sem_q)
        cp_kv = pltpu.async_copy(kvt.at[cols_v], kvbuf, sem_kv)
        cp_q.wait()
        cp_kv.wait()
        pltpu.sync_copy(qbuf, qe_out.at[pl.ds(base, C)])
        pltpu.sync_copy(kvbuf, kve_out.at[pl.ds(base, C)])
        return carry

    lax.fori_loop(0, cnt, chunk_body, 0)


def _gather_stage(qt, kvt, rows, cols):
    mesh = plsc.VectorSubcoreMesh(core_axis_name="c", subcore_axis_name="s",
                                  num_cores=NC, num_subcores=NS)
    f = pl.kernel(
        _gather_body,
        out_type=[
            jax.ShapeDtypeStruct((E, D), jnp.float32),
            jax.ShapeDtypeStruct((E, 2 * D), jnp.float32),
        ],
        mesh=mesh,
        scratch_types=(
            pltpu.VMEM((C,), jnp.int32),
            pltpu.VMEM((C,), jnp.int32),
            pltpu.VMEM((C, D), jnp.float32),
            pltpu.VMEM((C, 2 * D), jnp.float32),
            pltpu.SemaphoreType.DMA,
            pltpu.SemaphoreType.DMA,
        ),
        compiler_params=pltpu.CompilerParams(needs_layout_passes=False,
                                             use_tc_tiling_on_sc=False),
    )
    return f(qt, kvt, rows, cols)


def _edge_math_body(qe_ref, kve_ref, m1_ref, pay_ref):
    qe = qe_ref[...]
    ke = kve_ref[:, :D]
    ve = kve_ref[:, D:]
    m1 = m1_ref[...]
    s = jnp.dot(qe * ke, m1, preferred_element_type=jnp.float32)
    att = jnp.exp(jnp.clip(s, -10.0, 10.0))
    attb = jnp.dot(att, m1.T, preferred_element_type=jnp.float32)
    pay_ref[:, :D] = attb * ve
    pay_ref[:, D:D + H] = att
    pay_ref[:, D + H:] = jnp.zeros((EB, PW - D - H), jnp.float32)


def _edge_math(qe, kve, m1):
    grid = (E // EB,)
    return pl.pallas_call(
        _edge_math_body,
        grid=grid,
        in_specs=[
            pl.BlockSpec((EB, D), lambda i: (i, 0)),
            pl.BlockSpec((EB, 2 * D), lambda i: (i, 0)),
            pl.BlockSpec((D, H), lambda i: (0, 0)),
        ],
        out_specs=pl.BlockSpec((EB, PW), lambda i: (i, 0)),
        out_shape=jax.ShapeDtypeStruct((E, PW), jnp.float32),
    )(qe, kve, m1)


def _scatter_body(pay, rows_h, zacc,
                  acc_out,
                  rows_v, pbuf, acc_sh, sem_p):
    cid = lax.axis_index("c")
    sid = lax.axis_index("s")

    pltpu.sync_copy(zacc.at[pl.ds(sid * RPT, RPT)],
                    acc_sh.at[pl.ds(sid * RPT, RPT)])
    plsc.subcore_barrier()

    start = cid * CPC + sid * SBASE + jnp.minimum(sid, SEXTRA)
    cnt = SBASE + jnp.where(sid < SEXTRA, 1, 0)

    def chunk_body(j, carry):
        base = (start + j) * C
        pltpu.sync_copy(rows_h.at[pl.ds(base, C)], rows_v)
        cp = pltpu.async_copy(pay.at[pl.ds(base, C)], pbuf, sem_p)
        cp.wait()
        pltpu.sync_copy(pbuf, acc_sh.at[rows_v], add=True)
        return carry

    lax.fori_loop(0, cnt, chunk_body, 0)

    plsc.subcore_barrier()
    pltpu.sync_copy(acc_sh.at[pl.ds(sid * RPT, RPT)],
                    acc_out.at[cid, pl.ds(sid * RPT, RPT)])


def _scatter_stage(pay, rows, zacc):
    mesh = plsc.VectorSubcoreMesh(core_axis_name="c", subcore_axis_name="s",
                                  num_cores=NC, num_subcores=NS)
    f = pl.kernel(
        _scatter_body,
        out_type=jax.ShapeDtypeStruct((NC, NP, PW), jnp.float32),
        mesh=mesh,
        scratch_types=(
            pltpu.VMEM((C,), jnp.int32),
            pltpu.VMEM((C, PW), jnp.float32),
            pltpu.VMEM_SHARED((NP, PW), jnp.float32),
            pltpu.SemaphoreType.DMA,
        ),
        compiler_params=pltpu.CompilerParams(needs_layout_passes=False,
                                             use_tc_tiling_on_sc=False),
    )
    return f(pay, rows, zacc)


def _final_body(acc_ref, m2_ref, out_ref):
    acc = acc_ref[0] + acc_ref[1]
    den = jnp.dot(acc[:, D:D + H], m2_ref[...],
                  preferred_element_type=jnp.float32)
    out_ref[...] = acc[:, :D] / (den + 1e-8)


def _finalize(acc_p, m2):
    grid = (N // NB,)
    return pl.pallas_call(
        _final_body,
        grid=grid,
        in_specs=[
            pl.BlockSpec((NC, NB, PW), lambda i: (0, i, 0)),
            pl.BlockSpec((H, D), lambda i: (0, 0)),
        ],
        out_specs=pl.BlockSpec((NB, D), lambda i: (i, 0)),
        out_shape=jax.ShapeDtypeStruct((N, D), jnp.float32),
    )(acc_p, m2)


@jax.jit
def kernel(all_embeddings, edge_index, q, k, v):
    wkv = jnp.concatenate([k, v], axis=1)
    qt, kvt = _project(all_embeddings, q, wkv)
    rows = edge_index[0]
    cols = edge_index[1]
    qe, kve = _gather_stage(qt, kvt, rows, cols)
    m1 = jnp.repeat(jnp.eye(H, dtype=jnp.float32), HD, axis=0)
    pay = _edge_math(qe, kve, m1)
    zacc = jnp.zeros((NP, PW), jnp.float32)
    acc_p = _scatter_stage(pay, rows, zacc)
    m2 = jnp.repeat(jnp.eye(H, dtype=jnp.float32), HD, axis=1)
    return _finalize(acc_p, m2)


# speedup vs baseline: 2.7680x; 1.1228x over previous
import jax
import jax.numpy as jnp
from jax import lax
from jax.experimental import pallas as pl
from jax.experimental.pallas import tpu as pltpu
from jax.experimental.pallas import tpu_sc as plsc

N = 10000
E = 320000
D = 128
H = 8
HD = D // H

NC = 2
NS = 16

C = 128
W = NC * NS

NSLICE = 2
ES = E // NSLICE

NP = 10240
RPT = NP // NS
PW = D + 16

EB = 3200
NB = 400


def _proj_body(x_ref, wq_ref, wkv_ref, q_out, kv_out):
    x = x_ref[...]
    q_out[...] = jnp.dot(x, wq_ref[...], preferred_element_type=jnp.float32)
    kv_out[...] = jnp.dot(x, wkv_ref[...], preferred_element_type=jnp.float32)


def _project(emb, wq, wkv):
    grid = (N // NB,)
    return pl.pallas_call(
        _proj_body,
        grid=grid,
        in_specs=[
            pl.BlockSpec((NB, D), lambda i: (i, 0)),
            pl.BlockSpec((D, D), lambda i: (0, 0)),
            pl.BlockSpec((D, 2 * D), lambda i: (0, 0)),
        ],
        out_specs=[
            pl.BlockSpec((NB, D), lambda i: (i, 0)),
            pl.BlockSpec((NB, 2 * D), lambda i: (i, 0)),
        ],
        out_shape=[
            jax.ShapeDtypeStruct((N, D), jnp.float32),
            jax.ShapeDtypeStruct((N, 2 * D), jnp.float32),
        ],
    )(emb, wq, wkv)


def _gather_body(qt, kvt, rows_h, cols_h,
                 qe_out, kve_out,
                 rows_v, cols_v, qbuf, kvbuf, sem_q, sem_kv):
    chunks = ES // C
    gbase = chunks // W
    gextra = chunks - gbase * W

    cid = lax.axis_index("c")
    sid = lax.axis_index("s")
    w = cid * NS + sid
    start = w * gbase + jnp.minimum(w, gextra)
    cnt = gbase + jnp.where(w < gextra, 1, 0)

    def chunk_body(j, carry):
        base = (start + j) * C
        pltpu.sync_copy(rows_h.at[pl.ds(base, C)], rows_v)
        pltpu.sync_copy(cols_h.at[pl.ds(base, C)], cols_v)
        cp_q = pltpu.async_copy(qt.at[rows_v], qbuf, sem_q)
        cp_kv = pltpu.async_copy(kvt.at[cols_v], kvbuf, sem_kv)
        cp_q.wait()
        cp_kv.wait()
        pltpu.sync_copy(qbuf, qe_out.at[pl.ds(base, C)])
        pltpu.sync_copy(kvbuf, kve_out.at[pl.ds(base, C)])
        return carry

    lax.fori_loop(0, cnt, chunk_body, 0)


def _gather_stage(qt, kvt, rows, cols):
    mesh = plsc.VectorSubcoreMesh(core_axis_name="c", subcore_axis_name="s",
                                  num_cores=NC, num_subcores=NS)
    f = pl.kernel(
        _gather_body,
        out_type=[
            jax.ShapeDtypeStruct((ES, D), jnp.float32),
            jax.ShapeDtypeStruct((ES, 2 * D), jnp.float32),
        ],
        mesh=mesh,
        scratch_types=(
            pltpu.VMEM((C,), jnp.int32),
            pltpu.VMEM((C,), jnp.int32),
            pltpu.VMEM((C, D), jnp.float32),
            pltpu.VMEM((C, 2 * D), jnp.float32),
            pltpu.SemaphoreType.DMA,
            pltpu.SemaphoreType.DMA,
        ),
        compiler_params=pltpu.CompilerParams(needs_layout_passes=False,
                                             use_tc_tiling_on_sc=False),
    )
    return f(qt, kvt, rows, cols)


def _edge_math_body(qe_ref, kve_ref, m1_ref, pay_ref):
    qe = qe_ref[...]
    ke = kve_ref[:, :D]
    ve = kve_ref[:, D:]
    m1 = m1_ref[...]
    s = jnp.dot(qe * ke, m1, preferred_element_type=jnp.float32)
    att = jnp.exp(jnp.clip(s, -10.0, 10.0))
    attb = jnp.dot(att, m1.T, preferred_element_type=jnp.float32)
    pay_ref[:, :D] = attb * ve
    pay_ref[:, D:D + H] = att
    pay_ref[:, D + H:] = jnp.zeros((EB, PW - D - H), jnp.float32)


def _edge_math(qe, kve, m1):
    grid = (ES // EB,)
    return pl.pallas_call(
        _edge_math_body,
        grid=grid,
        in_specs=[
            pl.BlockSpec((EB, D), lambda i: (i, 0)),
            pl.BlockSpec((EB, 2 * D), lambda i: (i, 0)),
            pl.BlockSpec((D, H), lambda i: (0, 0)),
        ],
        out_specs=pl.BlockSpec((EB, PW), lambda i: (i, 0)),
        out_shape=jax.ShapeDtypeStruct((ES, PW), jnp.float32),
    )(qe, kve, m1)


def _scatter_body(pay, rows_h, zacc,
                  acc_out,
                  rows_v, pbuf, acc_sh, sem_p):
    chunks = ES // C
    cpc = chunks // NC
    sbase = cpc // NS
    sextra = cpc - sbase * NS

    cid = lax.axis_index("c")
    sid = lax.axis_index("s")

    pltpu.sync_copy(zacc.at[pl.ds(sid * RPT, RPT)],
                    acc_sh.at[pl.ds(sid * RPT, RPT)])
    plsc.subcore_barrier()

    start = cid * cpc + sid * sbase + jnp.minimum(sid, sextra)
    cnt = sbase + jnp.where(sid < sextra, 1, 0)

    def chunk_body(j, carry):
        base = (start + j) * C
        pltpu.sync_copy(rows_h.at[pl.ds(base, C)], rows_v)
        cp = pltpu.async_copy(pay.at[pl.ds(base, C)], pbuf, sem_p)
        cp.wait()
        pltpu.sync_copy(pbuf, acc_sh.at[rows_v], add=True)
        return carry

    lax.fori_loop(0, cnt, chunk_body, 0)

    plsc.subcore_barrier()
    pltpu.sync_copy(acc_sh.at[pl.ds(sid * RPT, RPT)],
                    acc_out.at[cid, pl.ds(sid * RPT, RPT)])


def _scatter_stage(pay, rows, zacc):
    mesh = plsc.VectorSubcoreMesh(core_axis_name="c", subcore_axis_name="s",
                                  num_cores=NC, num_subcores=NS)
    f = pl.kernel(
        _scatter_body,
        out_type=jax.ShapeDtypeStruct((NC, NP, PW), jnp.float32),
        mesh=mesh,
        scratch_types=(
            pltpu.VMEM((C,), jnp.int32),
            pltpu.VMEM((C, PW), jnp.float32),
            pltpu.VMEM_SHARED((NP, PW), jnp.float32),
            pltpu.SemaphoreType.DMA,
        ),
        compiler_params=pltpu.CompilerParams(needs_layout_passes=False,
                                             use_tc_tiling_on_sc=False),
    )
    return f(pay, rows, zacc)


def _final_body(acc_ref, m2_ref, out_ref):
    acc = acc_ref[0]
    for p in range(1, NSLICE * NC):
        acc = acc + acc_ref[p]
    den = jnp.dot(acc[:, D:D + H], m2_ref[...],
                  preferred_element_type=jnp.float32)
    out_ref[...] = acc[:, :D] / (den + 1e-8)


def _finalize(acc_p, m2):
    grid = (N // NB,)
    return pl.pallas_call(
        _final_body,
        grid=grid,
        in_specs=[
            pl.BlockSpec((NSLICE * NC, NB, PW), lambda i: (0, i, 0)),
            pl.BlockSpec((H, D), lambda i: (0, 0)),
        ],
        out_specs=pl.BlockSpec((NB, D), lambda i: (i, 0)),
        out_shape=jax.ShapeDtypeStruct((N, D), jnp.float32),
    )(acc_p, m2)


@jax.jit
def kernel(all_embeddings, edge_index, q, k, v):
    wkv = jnp.concatenate([k, v], axis=1)
    qt, kvt = _project(all_embeddings, q, wkv)
    rows = edge_index[0]
    cols = edge_index[1]
    m1 = jnp.repeat(jnp.eye(H, dtype=jnp.float32), HD, axis=0)
    zacc = jnp.zeros((NP, PW), jnp.float32)

    accs = []
    for s in range(NSLICE):
        r_s = lax.slice(rows, (s * ES,), ((s + 1) * ES,))
        c_s = lax.slice(cols, (s * ES,), ((s + 1) * ES,))
        qe, kve = _gather_stage(qt, kvt, r_s, c_s)
        pay = _edge_math(qe, kve, m1)
        accs.append(_scatter_stage(pay, r_s, zacc))

    acc_p = jnp.concatenate(accs, axis=0)
    m2 = jnp.repeat(jnp.eye(H, dtype=jnp.float32), HD, axis=1)
    return _finalize(acc_p, m2)
